# baseline (speedup 1.0000x reference)
import jax
import jax.numpy as jnp
from jax import lax
from jax.experimental import pallas as pl
from jax.experimental.pallas import tpu as pltpu
from jax.experimental.pallas import tpu_sc as plsc

N = 320000
FEAT = 128
NSEG = 10000

NC = 2
NS = 16
NW = NC * NS
RW = N // NW
CH = 40
NCHUNK = RW // CH
PLANE_R = 80
NSEG_PAD = 10240
SEG_PER_TILE = NSEG_PAD // NS


def _segsum_body(x_hbm, ids_hbm, zsum_hbm, osum_hbm, ocnt_hbm,
                 table, xb0, xb1, ib0, ib1, idsw, endt, startt,
                 sx0, sx1, si0, si1):
    cid = lax.axis_index("c")
    sid = lax.axis_index("s")
    wid = cid * NS + sid
    base = wid * RW

    r0 = sid * SEG_PER_TILE
    pltpu.sync_copy(zsum_hbm, xb0)
    for j in range(SEG_PER_TILE // CH):
        pltpu.sync_copy(xb0, table.at[pl.ds(r0 + j * CH, CH), :])
    plsc.subcore_barrier()

    pltpu.sync_copy(ids_hbm.at[pl.ds(base, RW)], idsw.at[pl.ds(16, RW)])
    lo = jnp.full((16,), -1, jnp.int32)
    hi = jnp.full((16,), 1 << 30, jnp.int32)
    idsw[pl.ds(0, 16)] = lo
    idsw[pl.ds(16 + RW, 16)] = hi
    for h in range(PLANE_R // CH):
        pltpu.sync_copy(zsum_hbm, endt.at[pl.ds(h * CH, CH), :])
        pltpu.sync_copy(zsum_hbm, startt.at[pl.ds(h * CH, CH), :])
    iota16 = jnp.arange(16, dtype=jnp.int32)

    def cbody(j, carry):
        k = j * 16
        cur = idsw[pl.ds(16 + k, 16)]
        nxt = idsw[pl.ds(17 + k, 16)]
        prv = idsw[pl.ds(15 + k, 16)]
        gpos = iota16 + k
        row = jax.lax.shift_right_logical(cur, 7)
        col = jax.lax.bitwise_and(cur, 127)
        plsc.store_scatter(endt, [row, col],
                           (gpos + 1).astype(jnp.float32), mask=cur != nxt)
        plsc.store_scatter(startt, [row, col],
                           gpos.astype(jnp.float32), mask=cur != prv)
        return carry

    lax.fori_loop(0, RW // 16, cbody, 0)
    c0 = wid * (2 * PLANE_R)
    pltpu.sync_copy(endt, ocnt_hbm.at[pl.ds(c0, PLANE_R), :])
    pltpu.sync_copy(startt, ocnt_hbm.at[pl.ds(c0 + PLANE_R, PLANE_R), :])

    def load(i, xb, ib):
        off = base + i * CH
        pltpu.sync_copy(x_hbm.at[pl.ds(off, CH), :], xb)
        pltpu.sync_copy(ids_hbm.at[pl.ds(off, CH)], ib)

    def sstart(xb, ib, sem):
        pltpu.async_copy(xb, table.at[ib], sem, add=True)

    def swait(xb, ib, sem):
        pltpu.make_async_copy(xb, table.at[ib], sem).wait()

    load(0, xb0, ib0)

    @pl.loop(0, NCHUNK // 2 - 1)
    def _loop(g):
        i0 = g * 2
        sstart(xb0, ib0, sx0)
        load(i0 + 1, xb1, ib1)
        swait(xb0, ib0, sx0)
        sstart(xb1, ib1, sx1)
        load(i0 + 2, xb0, ib0)
        swait(xb1, ib1, sx1)

    sstart(xb0, ib0, sx0)
    load(NCHUNK - 1, xb1, ib1)
    swait(xb0, ib0, sx0)
    sstart(xb1, ib1, sx1)
    swait(xb1, ib1, sx1)

    plsc.subcore_barrier()
    o0 = cid * NSEG_PAD + r0
    for j in range(SEG_PER_TILE // CH):
        pltpu.sync_copy(table.at[pl.ds(r0 + j * CH, CH), :], xb0)
        pltpu.sync_copy(xb0, osum_hbm.at[pl.ds(o0 + j * CH, CH), :])

_segsum = pl.kernel(
    _segsum_body,
    out_type=(
        jax.ShapeDtypeStruct((NC * NSEG_PAD, FEAT), jnp.float32),
        jax.ShapeDtypeStruct((NW * 2 * PLANE_R, FEAT), jnp.float32),
    ),
    mesh=plsc.VectorSubcoreMesh(core_axis_name="c", subcore_axis_name="s",
                                num_cores=NC, num_subcores=NS),
    compiler_params=pltpu.CompilerParams(needs_layout_passes=False),
    scratch_types=[
        pltpu.VMEM_SHARED((NSEG_PAD, FEAT), jnp.float32),
        pltpu.VMEM((CH, FEAT), jnp.float32),
        pltpu.VMEM((CH, FEAT), jnp.float32),
        pltpu.VMEM((CH,), jnp.int32),
        pltpu.VMEM((CH,), jnp.int32),
        pltpu.VMEM((RW + 32,), jnp.int32),
        pltpu.VMEM((PLANE_R, FEAT), jnp.float32),
        pltpu.VMEM((PLANE_R, FEAT), jnp.float32),
        pltpu.SemaphoreType.DMA,
        pltpu.SemaphoreType.DMA,
        pltpu.SemaphoreType.DMA,
        pltpu.SemaphoreType.DMA,
    ],
)


def _mlp_body(psum_ref, pcnt_ref, w1_ref, b1_ref, w2_ref, out_ref):
    sums = (psum_ref[:NSEG, :]
            + psum_ref[NSEG_PAD:NSEG_PAD + NSEG, :])
    cntp = jnp.zeros((PLANE_R, FEAT), jnp.float32)
    for w in range(NW):
        cntp = cntp + (pcnt_ref[w * 2 * PLANE_R:w * 2 * PLANE_R + PLANE_R, :]
                       - pcnt_ref[w * 2 * PLANE_R + PLANE_R:(w + 1) * 2 * PLANE_R, :])
    s_i = jax.lax.broadcasted_iota(jnp.int32, (NSEG, PLANE_R), 0)
    r_i = jax.lax.broadcasted_iota(jnp.int32, (NSEG, PLANE_R), 1)
    A = (jax.lax.shift_right_logical(s_i, 7) == r_i).astype(jnp.float32)
    rows = jax.lax.dot_general(A, cntp, (((1,), (0,)), ((), ())),
                               preferred_element_type=jnp.float32)
    s_j = jax.lax.broadcasted_iota(jnp.int32, (NSEG, FEAT), 0)
    c_j = jax.lax.broadcasted_iota(jnp.int32, (NSEG, FEAT), 1)
    B = (jax.lax.bitwise_and(s_j, 127) == c_j).astype(jnp.float32)
    cnt = jnp.sum(rows * B, axis=1, keepdims=True)
    means = sums / jnp.maximum(cnt, 1.0)
    h = jax.lax.dot_general(means, w1_ref[...],
                            (((1,), (1,)), ((), ())),
                            preferred_element_type=jnp.float32)
    h = jnp.tanh(h + b1_ref[...])
    scores = jax.lax.dot_general(h, w2_ref[...],
                                 (((1,), (1,)), ((), ())),
                                 preferred_element_type=jnp.float32)
    present = cnt > 0.0
    scores = jnp.where(present, scores, jnp.full_like(scores, -1e30))
    m = jnp.max(scores)
    e = jnp.exp(scores - m)
    out_ref[...] = e / jnp.sum(e)

_mlp = pl.pallas_call(
    _mlp_body,
    out_shape=jax.ShapeDtypeStruct((NSEG, 1), jnp.float32),
)


def _gather_body(attn_hbm, ids_hbm, out_hbm, table_v, ids_v, out_v):
    cid = lax.axis_index("c")
    sid = lax.axis_index("s")
    base = (cid * NS + sid) * RW
    pltpu.sync_copy(attn_hbm, table_v)
    pltpu.sync_copy(ids_hbm.at[pl.ds(base, RW)], ids_v)

    def body(j, carry):
        idx = ids_v[pl.ds(j * 16, 16)]
        out_v[pl.ds(j * 16, 16)] = plsc.load_gather(table_v, [idx])
        return carry

    lax.fori_loop(0, RW // 16, body, 0)
    pltpu.sync_copy(out_v, out_hbm.at[pl.ds(base, RW)])


_gather = pl.kernel(
    _gather_body,
    out_type=jax.ShapeDtypeStruct((N,), jnp.float32),
    mesh=plsc.VectorSubcoreMesh(core_axis_name="c", subcore_axis_name="s",
                                num_cores=NC, num_subcores=NS),
    compiler_params=pltpu.CompilerParams(needs_layout_passes=False),
    scratch_types=[
        pltpu.VMEM((NSEG,), jnp.float32),
        pltpu.VMEM((RW,), jnp.int32),
        pltpu.VMEM((RW,), jnp.float32),
    ],
)


def kernel(x, subst_ids, W1, b1, W2):
    ids = subst_ids.astype(jnp.int32)
    zsum = jnp.zeros((CH, FEAT), jnp.float32)
    psum, pcnt = _segsum(x, ids, zsum)
    attn = _mlp(psum, pcnt, W1, b1.reshape(1, 64), W2)
    out = _gather(attn.reshape(NSEG), ids)
    return out.reshape(N, 1)


# speedup vs baseline: 1.4886x; 1.4886x over previous
import jax
import jax.numpy as jnp
from jax import lax
from jax.experimental import pallas as pl
from jax.experimental.pallas import tpu as pltpu
from jax.experimental.pallas import tpu_sc as plsc

N = 320000
FEAT = 128
NSEG = 10000

NC = 2
NS = 16
NW = NC * NS
RW = N // NW
CH = 80
NCHUNK = RW // CH
PLANE_R = 80
NSEG_PAD = 10240
SEG_PER_TILE = NSEG_PAD // NS


def _segsum_body(x_hbm, ids_hbm, osum_hbm, ocnt_hbm,
                 table, xb0, xb1, ib0, ib1, idsw, endt, startt,
                 sx0, sx1, si0, si1):
    cid = lax.axis_index("c")
    sid = lax.axis_index("s")
    wid = cid * NS + sid
    base = wid * RW

    r0 = sid * SEG_PER_TILE
    z16 = jnp.zeros((16,), jnp.float32)

    def zbody(j, carry):
        r = jax.lax.shift_right_logical(j, 3)
        c = jax.lax.bitwise_and(j, 7) * 16
        xb0[r, pl.ds(c, 16)] = z16
        endt[r, pl.ds(c, 16)] = z16
        startt[r, pl.ds(c, 16)] = z16
        return carry

    lax.fori_loop(0, PLANE_R * (FEAT // 16), zbody, 0)
    for j in range(SEG_PER_TILE // CH):
        pltpu.sync_copy(xb0, table.at[pl.ds(r0 + j * CH, CH), :])
    plsc.subcore_barrier()

    pltpu.sync_copy(ids_hbm.at[pl.ds(base, RW)], idsw.at[pl.ds(16, RW)])
    lo = jnp.full((16,), -1, jnp.int32)
    hi = jnp.full((16,), 1 << 30, jnp.int32)
    idsw[pl.ds(0, 16)] = lo
    idsw[pl.ds(16 + RW, 16)] = hi
    iota16 = jnp.arange(16, dtype=jnp.int32)

    def cbody(j, carry):
        k = j * 16
        cur = idsw[pl.ds(16 + k, 16)]
        nxt = idsw[pl.ds(17 + k, 16)]
        prv = idsw[pl.ds(15 + k, 16)]
        gpos = iota16 + k
        row = jax.lax.shift_right_logical(cur, 7)
        col = jax.lax.bitwise_and(cur, 127)
        plsc.store_scatter(endt, [row, col],
                           (gpos + 1).astype(jnp.float32), mask=cur != nxt)
        plsc.store_scatter(startt, [row, col],
                           gpos.astype(jnp.float32), mask=cur != prv)
        return carry

    lax.fori_loop(0, RW // 16, cbody, 0)

    def sbody(j, carry):
        r = jax.lax.shift_right_logical(j, 3)
        c = jax.lax.bitwise_and(j, 7) * 16
        endt[r, pl.ds(c, 16)] = endt[r, pl.ds(c, 16)] - startt[r, pl.ds(c, 16)]
        return carry

    lax.fori_loop(0, PLANE_R * (FEAT // 16), sbody, 0)
    pltpu.sync_copy(endt, ocnt_hbm.at[pl.ds(wid * PLANE_R, PLANE_R), :])

    @pl.loop(0, NCHUNK)
    def _loop(i):
        off = base + i * CH
        pltpu.sync_copy(x_hbm.at[pl.ds(off, CH), :], xb0)
        for q in range(CH // 16):
            ib0[pl.ds(q * 16, 16)] = idsw[pl.ds(16 + i * CH + q * 16, 16)]
        pltpu.sync_copy(xb0, table.at[ib0], add=True)

    plsc.subcore_barrier()
    o0 = cid * NSEG_PAD + r0
    for j in range(SEG_PER_TILE // CH):
        pltpu.sync_copy(table.at[pl.ds(r0 + j * CH, CH), :], xb0)
        pltpu.sync_copy(xb0, osum_hbm.at[pl.ds(o0 + j * CH, CH), :])

_segsum = pl.kernel(
    _segsum_body,
    out_type=(
        jax.ShapeDtypeStruct((NC * NSEG_PAD, FEAT), jnp.float32),
        jax.ShapeDtypeStruct((NW * PLANE_R, FEAT), jnp.float32),
    ),
    mesh=plsc.VectorSubcoreMesh(core_axis_name="c", subcore_axis_name="s",
                                num_cores=NC, num_subcores=NS),
    compiler_params=pltpu.CompilerParams(needs_layout_passes=False),
    scratch_types=[
        pltpu.VMEM_SHARED((NSEG_PAD, FEAT), jnp.float32),
        pltpu.VMEM((CH, FEAT), jnp.float32),
        pltpu.VMEM((CH, FEAT), jnp.float32),
        pltpu.VMEM((CH,), jnp.int32),
        pltpu.VMEM((CH,), jnp.int32),
        pltpu.VMEM((RW + 32,), jnp.int32),
        pltpu.VMEM((PLANE_R, FEAT), jnp.float32),
        pltpu.VMEM((PLANE_R, FEAT), jnp.float32),
        pltpu.SemaphoreType.DMA,
        pltpu.SemaphoreType.DMA,
        pltpu.SemaphoreType.DMA,
        pltpu.SemaphoreType.DMA,
    ],
)


def _mlp_body(psum_ref, pcnt_ref, w1_ref, b1_ref, w2_ref, out_ref):
    sums = (psum_ref[:NSEG, :]
            + psum_ref[NSEG_PAD:NSEG_PAD + NSEG, :])
    cntp = jnp.zeros((PLANE_R, FEAT), jnp.float32)
    for w in range(NW):
        cntp = cntp + pcnt_ref[w * PLANE_R:(w + 1) * PLANE_R, :]
    s_i = jax.lax.broadcasted_iota(jnp.int32, (NSEG, PLANE_R), 0)
    r_i = jax.lax.broadcasted_iota(jnp.int32, (NSEG, PLANE_R), 1)
    A = (jax.lax.shift_right_logical(s_i, 7) == r_i).astype(jnp.float32)
    rows = jax.lax.dot_general(A, cntp, (((1,), (0,)), ((), ())),
                               preferred_element_type=jnp.float32)
    s_j = jax.lax.broadcasted_iota(jnp.int32, (NSEG, FEAT), 0)
    c_j = jax.lax.broadcasted_iota(jnp.int32, (NSEG, FEAT), 1)
    B = (jax.lax.bitwise_and(s_j, 127) == c_j).astype(jnp.float32)
    cnt = jnp.sum(rows * B, axis=1, keepdims=True)
    means = sums / jnp.maximum(cnt, 1.0)
    h = jax.lax.dot_general(means, w1_ref[...],
                            (((1,), (1,)), ((), ())),
                            preferred_element_type=jnp.float32)
    h = jnp.tanh(h + b1_ref[...])
    scores = jax.lax.dot_general(h, w2_ref[...],
                                 (((1,), (1,)), ((), ())),
                                 preferred_element_type=jnp.float32)
    present = cnt > 0.0
    scores = jnp.where(present, scores, jnp.full_like(scores, -1e30))
    m = jnp.max(scores)
    e = jnp.exp(scores - m)
    out_ref[...] = e / jnp.sum(e)

_mlp = pl.pallas_call(
    _mlp_body,
    out_shape=jax.ShapeDtypeStruct((NSEG, 1), jnp.float32),
)


def _gather_body(attn_hbm, ids_hbm, out_hbm, table_v, ids_v, out_v):
    cid = lax.axis_index("c")
    sid = lax.axis_index("s")
    base = (cid * NS + sid) * RW
    pltpu.sync_copy(attn_hbm, table_v)
    pltpu.sync_copy(ids_hbm.at[pl.ds(base, RW)], ids_v)

    def body(j, carry):
        idx = ids_v[pl.ds(j * 16, 16)]
        out_v[pl.ds(j * 16, 16)] = plsc.load_gather(table_v, [idx])
        return carry

    lax.fori_loop(0, RW // 16, body, 0)
    pltpu.sync_copy(out_v, out_hbm.at[pl.ds(base, RW)])


_gather = pl.kernel(
    _gather_body,
    out_type=jax.ShapeDtypeStruct((N,), jnp.float32),
    mesh=plsc.VectorSubcoreMesh(core_axis_name="c", subcore_axis_name="s",
                                num_cores=NC, num_subcores=NS),
    compiler_params=pltpu.CompilerParams(needs_layout_passes=False),
    scratch_types=[
        pltpu.VMEM((NSEG,), jnp.float32),
        pltpu.VMEM((RW,), jnp.int32),
        pltpu.VMEM((RW,), jnp.float32),
    ],
)


def kernel(x, subst_ids, W1, b1, W2):
    ids = subst_ids.astype(jnp.int32)
    psum, pcnt = _segsum(x, ids)
    attn = _mlp(psum, pcnt, W1, b1.reshape(1, 64), W2)
    out = _gather(attn.reshape(NSEG), ids)
    return out.reshape(N, 1)
